# Initial kernel scaffold; baseline (speedup 1.0000x reference)
import jax
import jax.numpy as jnp
from jax import lax
from jax.experimental import pallas as pl
from jax.experimental.pallas import tpu as pltpu
from jax.experimental.pallas import tpu_sc as plsc

N = 10000
E = 320000
D = 128
DOUT = 128

NC = 2
NS = 16
NW = NC * NS
EPW = E // NW
C = 80
K = EPW // C
U = 5
NPAD = 10240
RPT = NPAD // NS


def _sc_body(h_hbm, src_hbm, dst_hbm, sums_hbm, cnts_hbm,
             acc_sh, cnt_sh, srcs_v, dsts_v, rows_v, ones_v, zrow_v, zcnt_v, sem):
  cid = lax.axis_index("c")
  sid = lax.axis_index("s")
  w = cid * NS + sid

  @pl.loop(0, C // 16)
  def _(i):
    ones_v[pl.ds(i * 16, 16)] = jnp.ones((16,), jnp.float32)

  @pl.loop(0, C)
  def _(r):
    for u in range(D // 16):
      zrow_v[r, pl.ds(u * 16, 16)] = jnp.zeros((16,), jnp.float32)

  @pl.loop(0, RPT // 16)
  def _(i):
    zcnt_v[pl.ds(i * 16, 16)] = jnp.zeros((16,), jnp.float32)

  row0 = sid * RPT
  for t in range(RPT // C):
    pltpu.sync_copy(zrow_v, acc_sh.at[pl.ds(row0 + t * C, C)])
  pltpu.sync_copy(zcnt_v, cnt_sh.at[pl.ds(row0, RPT)])

  pltpu.sync_copy(src_hbm.at[w], srcs_v)
  pltpu.sync_copy(dst_hbm.at[w], dsts_v)

  plsc.subcore_barrier()

  @pl.loop(0, K, step=U)
  def _(j0):
    for u in range(U):
      j = j0 + u
      pltpu.async_copy(h_hbm.at[srcs_v.at[j]], rows_v, sem).wait()
      pltpu.sync_copy(rows_v, acc_sh.at[dsts_v.at[j]], add=True)
      pltpu.sync_copy(ones_v, cnt_sh.at[dsts_v.at[j]], add=True)

  plsc.subcore_barrier()

  pltpu.sync_copy(acc_sh.at[pl.ds(row0, RPT)], sums_hbm.at[cid, pl.ds(row0, RPT)])
  pltpu.sync_copy(cnt_sh.at[pl.ds(row0, RPT)], cnts_hbm.at[cid, pl.ds(row0, RPT)])


def _sc_aggregate(h, src, dst):
  mesh = plsc.VectorSubcoreMesh(core_axis_name="c", subcore_axis_name="s")
  return pl.kernel(
      _sc_body,
      mesh=mesh,
      out_type=[
          jax.ShapeDtypeStruct((NC, NPAD, D), jnp.float32),
          jax.ShapeDtypeStruct((NC, NPAD), jnp.float32),
      ],
      scratch_types=[
          pltpu.VMEM_SHARED((NPAD, D), jnp.float32),
          pltpu.VMEM_SHARED((NPAD,), jnp.float32),
          pltpu.VMEM((K, C), jnp.int32),
          pltpu.VMEM((K, C), jnp.int32),
          pltpu.VMEM((C, D), jnp.float32),
          pltpu.VMEM((C,), jnp.float32),
          pltpu.VMEM((C, D), jnp.float32),
          pltpu.VMEM((RPT,), jnp.float32),
          pltpu.SemaphoreType.DMA,
      ],
  )(h, src, dst)


def _tc_body(h_ref, s_ref, c_ref, w_ref, b_ref, o_ref):
  cnt = jnp.maximum(c_ref[0] + c_ref[1], 1.0)
  h_n = (s_ref[0] + s_ref[1]) / cnt
  dn = (((1,), (1,)), ((), ()))
  self_part = lax.dot_general(h_ref[...], w_ref[:, 0:D], dn,
                              preferred_element_type=jnp.float32)
  neigh_part = lax.dot_general(h_n, w_ref[:, D:2 * D], dn,
                               preferred_element_type=jnp.float32)
  o_ref[...] = self_part + neigh_part + b_ref[...]


def _tc_finish(h, sums, cnts, W, b):
  B = 1000
  return pl.pallas_call(
      _tc_body,
      grid=(N // B,),
      in_specs=[
          pl.BlockSpec((B, D), lambda i: (i, 0)),
          pl.BlockSpec((NC, B, D), lambda i: (0, i, 0)),
          pl.BlockSpec((NC, B, 1), lambda i: (0, i, 0)),
          pl.BlockSpec((DOUT, 2 * D), lambda i: (0, 0)),
          pl.BlockSpec((1, DOUT), lambda i: (0, 0)),
      ],
      out_specs=pl.BlockSpec((B, DOUT), lambda i: (i, 0)),
      out_shape=jax.ShapeDtypeStruct((N, DOUT), jnp.float32),
  )(h, sums, cnts, W, b)


@jax.jit
def kernel(h, edge_index, W, b):
  src = edge_index[0].reshape(NW, K, C)
  dst = edge_index[1].reshape(NW, K, C)
  sums, cnts = _sc_aggregate(h, src, dst)
  return _tc_finish(h, sums, cnts.reshape(NC, NPAD, 1), W, b.reshape(1, DOUT))


# speedup vs baseline: 8.4462x; 8.4462x over previous
import jax
import jax.numpy as jnp
from jax import lax
from jax.experimental import pallas as pl
from jax.experimental.pallas import tpu as pltpu
from jax.experimental.pallas import tpu_sc as plsc

N = 10000
E = 320000
D = 128
DOUT = 128

NC = 2
NS = 16
NW = NC * NS
EPW = E // NW
C = 80
K = EPW // C
U = 5
NPAD = 10240
RPT = NPAD // NS


def _sc_body(h_hbm, src_hbm, dst_hbm, sums_hbm, cnts_hbm,
             acc_sh, cnt_sh, srcs_v, dsts_v, rows_v, ones_v, zcnt_v, sem):
  cid = lax.axis_index("c")
  sid = lax.axis_index("s")
  w = cid * NS + sid

  @pl.loop(0, C // 16)
  def _(i):
    ones_v[pl.ds(i * 16, 16)] = jnp.ones((16,), jnp.float32)
    zcnt_v[pl.ds(i * 16, 16)] = jnp.zeros((16,), jnp.float32)

  @pl.loop(0, C)
  def _(r):
    for u in range(D // 16):
      rows_v[r, pl.ds(u * 16, 16)] = jnp.zeros((16,), jnp.float32)

  row0 = sid * RPT
  for t in range(RPT // C):
    pltpu.sync_copy(rows_v, acc_sh.at[pl.ds(row0 + t * C, C)])
    pltpu.sync_copy(zcnt_v, cnt_sh.at[pl.ds(row0 + t * C, C)])

  pltpu.sync_copy(src_hbm.at[w], srcs_v)
  pltpu.sync_copy(dst_hbm.at[w], dsts_v)

  plsc.subcore_barrier()

  @pl.loop(0, K, step=U)
  def _(j0):
    for u in range(U):
      j = j0 + u
      pltpu.async_copy(h_hbm.at[srcs_v.at[j]], rows_v, sem).wait()
      pltpu.sync_copy(rows_v, acc_sh.at[dsts_v.at[j]], add=True)
      pltpu.sync_copy(ones_v, cnt_sh.at[dsts_v.at[j]], add=True)

  plsc.subcore_barrier()

  pltpu.sync_copy(acc_sh.at[pl.ds(row0, RPT)], sums_hbm.at[cid, pl.ds(row0, RPT)])
  pltpu.sync_copy(cnt_sh.at[pl.ds(row0, RPT)], cnts_hbm.at[cid, pl.ds(row0, RPT)])


def _sc_aggregate(h, src, dst):
  mesh = plsc.VectorSubcoreMesh(core_axis_name="c", subcore_axis_name="s")
  return pl.kernel(
      _sc_body,
      mesh=mesh,
      out_type=[
          jax.ShapeDtypeStruct((NC, NPAD, D), jnp.float32),
          jax.ShapeDtypeStruct((NC, NPAD), jnp.float32),
      ],
      scratch_types=[
          pltpu.VMEM_SHARED((NPAD, D), jnp.float32),
          pltpu.VMEM_SHARED((NPAD,), jnp.float32),
          pltpu.VMEM((K, C), jnp.int32),
          pltpu.VMEM((K, C), jnp.int32),
          pltpu.VMEM((C, D), jnp.float32),
          pltpu.VMEM((C,), jnp.float32),
          pltpu.VMEM((C,), jnp.float32),
          pltpu.SemaphoreType.DMA,
      ],
  )(h, src, dst)


def _tc_body(h_ref, s_ref, c_ref, w_ref, b_ref, o_ref):
  cnt = jnp.maximum(c_ref[0] + c_ref[1], 1.0)
  h_n = (s_ref[0] + s_ref[1]) / cnt
  dn = (((1,), (1,)), ((), ()))
  self_part = lax.dot_general(h_ref[...], w_ref[:, 0:D], dn,
                              preferred_element_type=jnp.float32)
  neigh_part = lax.dot_general(h_n, w_ref[:, D:2 * D], dn,
                               preferred_element_type=jnp.float32)
  o_ref[...] = self_part + neigh_part + b_ref[...]


def _tc_finish(h, sums, cnts, W, b):
  B = 1000
  return pl.pallas_call(
      _tc_body,
      grid=(N // B,),
      in_specs=[
          pl.BlockSpec((B, D), lambda i: (i, 0)),
          pl.BlockSpec((NC, B, D), lambda i: (0, i, 0)),
          pl.BlockSpec((NC, B, 1), lambda i: (0, i, 0)),
          pl.BlockSpec((DOUT, 2 * D), lambda i: (0, 0)),
          pl.BlockSpec((1, DOUT), lambda i: (0, 0)),
      ],
      out_specs=pl.BlockSpec((B, DOUT), lambda i: (i, 0)),
      out_shape=jax.ShapeDtypeStruct((N, DOUT), jnp.float32),
  )(h, sums, cnts, W, b)


@jax.jit
def kernel(h, edge_index, W, b):
  src = edge_index[0].reshape(NW, K, C)
  dst = edge_index[1].reshape(NW, K, C)
  sums, cnts = _sc_aggregate(h, src, dst)
  return _tc_finish(h, sums, cnts.reshape(NC, NPAD, 1), W, b.reshape(1, DOUT))


# speedup vs baseline: 10.9936x; 1.3016x over previous
import jax
import jax.numpy as jnp
from jax import lax
from jax.experimental import pallas as pl
from jax.experimental.pallas import tpu as pltpu
from jax.experimental.pallas import tpu_sc as plsc

N = 10000
E = 320000
D = 128
DOUT = 128

NC = 2
NS = 16
NW = NC * NS
EPW = E // NW
C = 80
K = EPW // C
NPAD = 10240
RPT = NPAD // NS


def _sc_body(h_hbm, src_hbm, dst_hbm, sums_hbm, cnts_hbm,
             acc_sh, cnt_sh, srcs_v, dsts_v, rows0_v, rows1_v, ones_v, zcnt_v,
             sem0, sem1):
  cid = lax.axis_index("c")
  sid = lax.axis_index("s")
  w = cid * NS + sid
  rows = (rows0_v, rows1_v)
  sems = (sem0, sem1)

  @pl.loop(0, C // 16)
  def _(i):
    ones_v[pl.ds(i * 16, 16)] = jnp.ones((16,), jnp.float32)
    zcnt_v[pl.ds(i * 16, 16)] = jnp.zeros((16,), jnp.float32)

  @pl.loop(0, C)
  def _(r):
    for u in range(D // 16):
      rows0_v[r, pl.ds(u * 16, 16)] = jnp.zeros((16,), jnp.float32)

  row0 = sid * RPT
  for t in range(RPT // C):
    pltpu.sync_copy(rows0_v, acc_sh.at[pl.ds(row0 + t * C, C)])
    pltpu.sync_copy(zcnt_v, cnt_sh.at[pl.ds(row0 + t * C, C)])

  pltpu.sync_copy(src_hbm.at[w], srcs_v)
  pltpu.sync_copy(dst_hbm.at[w], dsts_v)

  plsc.subcore_barrier()

  def gather_start(j, b):
    pltpu.async_copy(h_hbm.at[srcs_v.at[pl.ds(j * C, C)]], rows[b], sems[b])

  def gather_wait(j, b):
    pltpu.make_async_copy(
        h_hbm.at[srcs_v.at[pl.ds(j * C, C)]], rows[b], sems[b]).wait()

  def scatter(j, b):
    pltpu.sync_copy(rows[b], acc_sh.at[dsts_v.at[j]], add=True)
    pltpu.sync_copy(ones_v, cnt_sh.at[dsts_v.at[j]], add=True)

  gather_start(0, 0)

  @pl.loop(0, K - 1, step=2)
  def _(j0):
    for u in range(2):
      j = j0 + u
      gather_wait(j, u)
      gather_start(j + 1, 1 - u)
      scatter(j, u)

  gather_wait(K - 1, 0)
  scatter(K - 1, 0)

  plsc.subcore_barrier()

  pltpu.sync_copy(acc_sh.at[pl.ds(row0, RPT)], sums_hbm.at[cid, pl.ds(row0, RPT)])
  pltpu.sync_copy(cnt_sh.at[pl.ds(row0, RPT)], cnts_hbm.at[cid, pl.ds(row0, RPT)])


def _sc_aggregate(h, src, dst):
  mesh = plsc.VectorSubcoreMesh(core_axis_name="c", subcore_axis_name="s")
  return pl.kernel(
      _sc_body,
      mesh=mesh,
      out_type=[
          jax.ShapeDtypeStruct((NC, NPAD, D), jnp.float32),
          jax.ShapeDtypeStruct((NC, NPAD), jnp.float32),
      ],
      scratch_types=[
          pltpu.VMEM_SHARED((NPAD, D), jnp.float32),
          pltpu.VMEM_SHARED((NPAD,), jnp.float32),
          pltpu.VMEM((EPW,), jnp.int32),
          pltpu.VMEM((K, C), jnp.int32),
          pltpu.VMEM((C, D), jnp.float32),
          pltpu.VMEM((C, D), jnp.float32),
          pltpu.VMEM((80,), jnp.float32),
          pltpu.VMEM((80,), jnp.float32),
          pltpu.SemaphoreType.DMA,
          pltpu.SemaphoreType.DMA,
      ],
  )(h, src, dst)


def _tc_body(h_ref, s_ref, c_ref, w_ref, b_ref, o_ref):
  cnt = jnp.maximum(c_ref[0] + c_ref[1], 1.0)
  h_n = (s_ref[0] + s_ref[1]) / cnt
  dn = (((1,), (1,)), ((), ()))
  self_part = lax.dot_general(h_ref[...], w_ref[:, 0:D], dn,
                              preferred_element_type=jnp.float32)
  neigh_part = lax.dot_general(h_n, w_ref[:, D:2 * D], dn,
                               preferred_element_type=jnp.float32)
  o_ref[...] = self_part + neigh_part + b_ref[...]


def _tc_finish(h, sums, cnts, W, b):
  B = 1000
  return pl.pallas_call(
      _tc_body,
      grid=(N // B,),
      in_specs=[
          pl.BlockSpec((B, D), lambda i: (i, 0)),
          pl.BlockSpec((NC, B, D), lambda i: (0, i, 0)),
          pl.BlockSpec((NC, B, 1), lambda i: (0, i, 0)),
          pl.BlockSpec((DOUT, 2 * D), lambda i: (0, 0)),
          pl.BlockSpec((1, DOUT), lambda i: (0, 0)),
      ],
      out_specs=pl.BlockSpec((B, DOUT), lambda i: (i, 0)),
      out_shape=jax.ShapeDtypeStruct((N, DOUT), jnp.float32),
  )(h, sums, cnts, W, b)


@jax.jit
def kernel(h, edge_index, W, b):
  src = edge_index[0].reshape(NW, EPW)
  dst = edge_index[1].reshape(NW, K, C)
  sums, cnts = _sc_aggregate(h, src, dst)
  return _tc_finish(h, sums, cnts.reshape(NC, NPAD, 1), W, b.reshape(1, DOUT))


# speedup vs baseline: 11.1031x; 1.0100x over previous
import jax
import jax.numpy as jnp
from jax import lax
from jax.experimental import pallas as pl
from jax.experimental.pallas import tpu as pltpu
from jax.experimental.pallas import tpu_sc as plsc

N = 10000
E = 320000
D = 128
DOUT = 128

NC = 2
NS = 16
NW = NC * NS
EPW = E // NW
C = 80
K = EPW // C
NPAD = 10240
RPT = NPAD // NS


def _sc_body(h_hbm, src_hbm, dst_hbm, sums_hbm, cnts_hbm,
             acc_sh, cnt_sh, srcs_v, dsts_v, rows0_v, rows1_v, ones_v, zcnt_v,
             sem0, sem1, ssem0, ssem1):
  cid = lax.axis_index("c")
  sid = lax.axis_index("s")
  w = cid * NS + sid
  rows = (rows0_v, rows1_v)
  sems = (sem0, sem1)
  ssems = (ssem0, ssem1)

  @pl.loop(0, C // 16)
  def _(i):
    ones_v[pl.ds(i * 16, 16)] = jnp.ones((16,), jnp.float32)
    zcnt_v[pl.ds(i * 16, 16)] = jnp.zeros((16,), jnp.float32)

  @pl.loop(0, C)
  def _(r):
    for u in range(D // 16):
      rows0_v[r, pl.ds(u * 16, 16)] = jnp.zeros((16,), jnp.float32)

  row0 = sid * RPT
  for t in range(RPT // C):
    pltpu.sync_copy(rows0_v, acc_sh.at[pl.ds(row0 + t * C, C)])
    pltpu.sync_copy(zcnt_v, cnt_sh.at[pl.ds(row0 + t * C, C)])

  pltpu.sync_copy(src_hbm.at[w], srcs_v)
  pltpu.sync_copy(dst_hbm.at[w], dsts_v)

  plsc.subcore_barrier()

  def gather_start(j, b):
    pltpu.async_copy(h_hbm.at[srcs_v.at[pl.ds(j * C, C)]], rows[b], sems[b])

  def gather_wait(j, b):
    pltpu.make_async_copy(
        h_hbm.at[srcs_v.at[pl.ds(j * C, C)]], rows[b], sems[b]).wait()

  def scatter_start(j, b):
    pltpu.async_copy(rows[b], acc_sh.at[dsts_v.at[j]], ssems[b], add=True)

  def scatter_wait(j, b):
    pltpu.make_async_copy(rows[b], acc_sh.at[dsts_v.at[j]], ssems[b]).wait()

  def counts(j):
    pltpu.sync_copy(ones_v, cnt_sh.at[dsts_v.at[j]], add=True)

  gather_start(0, 0)
  gather_wait(0, 0)
  scatter_start(0, 0)
  gather_start(1, 1)
  counts(0)
  gather_wait(1, 1)
  scatter_start(1, 1)
  scatter_wait(0, 0)
  gather_start(2, 0)
  counts(1)

  @pl.loop(2, K - 1, step=2)
  def _(j0):
    for u in range(2):
      j = j0 + u
      gather_wait(j, u)
      scatter_start(j, u)
      scatter_wait(j - 1, 1 - u)
      gather_start(j + 1, 1 - u)
      counts(j)

  gather_wait(K - 1, 0)
  scatter_start(K - 1, 0)
  scatter_wait(K - 2, 1)
  counts(K - 1)
  scatter_wait(K - 1, 0)

  plsc.subcore_barrier()

  pltpu.sync_copy(acc_sh.at[pl.ds(row0, RPT)], sums_hbm.at[cid, pl.ds(row0, RPT)])
  pltpu.sync_copy(cnt_sh.at[pl.ds(row0, RPT)], cnts_hbm.at[cid, pl.ds(row0, RPT)])


def _sc_aggregate(h, src, dst):
  mesh = plsc.VectorSubcoreMesh(core_axis_name="c", subcore_axis_name="s")
  return pl.kernel(
      _sc_body,
      mesh=mesh,
      out_type=[
          jax.ShapeDtypeStruct((NC, NPAD, D), jnp.float32),
          jax.ShapeDtypeStruct((NC, NPAD), jnp.float32),
      ],
      scratch_types=[
          pltpu.VMEM_SHARED((NPAD, D), jnp.float32),
          pltpu.VMEM_SHARED((NPAD,), jnp.float32),
          pltpu.VMEM((EPW,), jnp.int32),
          pltpu.VMEM((K, C), jnp.int32),
          pltpu.VMEM((C, D), jnp.float32),
          pltpu.VMEM((C, D), jnp.float32),
          pltpu.VMEM((80,), jnp.float32),
          pltpu.VMEM((80,), jnp.float32),
          pltpu.SemaphoreType.DMA,
          pltpu.SemaphoreType.DMA,
          pltpu.SemaphoreType.DMA,
          pltpu.SemaphoreType.DMA,
      ],
  )(h, src, dst)


def _tc_body(h_ref, s_ref, c_ref, w_ref, b_ref, o_ref):
  cnt = jnp.maximum(c_ref[0] + c_ref[1], 1.0)
  h_n = (s_ref[0] + s_ref[1]) / cnt
  dn = (((1,), (1,)), ((), ()))
  self_part = lax.dot_general(h_ref[...], w_ref[:, 0:D], dn,
                              preferred_element_type=jnp.float32)
  neigh_part = lax.dot_general(h_n, w_ref[:, D:2 * D], dn,
                               preferred_element_type=jnp.float32)
  o_ref[...] = self_part + neigh_part + b_ref[...]


def _tc_finish(h, sums, cnts, W, b):
  B = 2000
  return pl.pallas_call(
      _tc_body,
      grid=(N // B,),
      in_specs=[
          pl.BlockSpec((B, D), lambda i: (i, 0)),
          pl.BlockSpec((NC, B, D), lambda i: (0, i, 0)),
          pl.BlockSpec((NC, B, 1), lambda i: (0, i, 0)),
          pl.BlockSpec((DOUT, 2 * D), lambda i: (0, 0)),
          pl.BlockSpec((1, DOUT), lambda i: (0, 0)),
      ],
      out_specs=pl.BlockSpec((B, DOUT), lambda i: (i, 0)),
      out_shape=jax.ShapeDtypeStruct((N, DOUT), jnp.float32),
  )(h, sums, cnts, W, b)


@jax.jit
def kernel(h, edge_index, W, b):
  src = edge_index[0].reshape(NW, EPW)
  dst = edge_index[1].reshape(NW, K, C)
  sums, cnts = _sc_aggregate(h, src, dst)
  return _tc_finish(h, sums, cnts.reshape(NC, NPAD, 1), W, b.reshape(1, DOUT))


# speedup vs baseline: 11.9773x; 1.0787x over previous
import jax
import jax.numpy as jnp
from jax import lax
from jax.experimental import pallas as pl
from jax.experimental.pallas import tpu as pltpu
from jax.experimental.pallas import tpu_sc as plsc

N = 10000
E = 320000
D = 128
DOUT = 128

NC = 2
NS = 16
NW = NC * NS
EPW = E // NW
C = 80
K = EPW // C
NPAD = 10240
RPT = NPAD // NS


def _sc_body(h_hbm, src_hbm, dst_hbm, sums_hbm, cnts_hbm,
             acc_sh, cnt_sh, srcs_v, dsts_v, rows0_v, rows1_v, ones_v, zcnt_v,
             sem0, sem1, gsem0, gsem1, ssem0, ssem1):
  cid = lax.axis_index("c")
  sid = lax.axis_index("s")
  w = cid * NS + sid
  rows = (rows0_v, rows1_v)
  sems = (sem0, sem1)
  gsems = (gsem0, gsem1)
  ssems = (ssem0, ssem1)

  @pl.loop(0, C // 16)
  def _(i):
    ones_v[pl.ds(i * 16, 16)] = jnp.ones((16,), jnp.float32)
    zcnt_v[pl.ds(i * 16, 16)] = jnp.zeros((16,), jnp.float32)

  @pl.loop(0, C)
  def _(r):
    for u in range(D // 16):
      rows0_v[r, pl.ds(u * 16, 16)] = jnp.zeros((16,), jnp.float32)

  row0 = sid * RPT
  for t in range(RPT // C):
    pltpu.sync_copy(rows0_v, acc_sh.at[pl.ds(row0 + t * C, C)])
    pltpu.sync_copy(zcnt_v, cnt_sh.at[pl.ds(row0 + t * C, C)])

  pltpu.sync_copy(src_hbm.at[w], srcs_v)
  pltpu.sync_copy(dst_hbm.at[w], dsts_v)

  plsc.subcore_barrier()

  H = C // 2

  def gather_start(j, b):
    pltpu.async_copy(h_hbm.at[srcs_v.at[pl.ds(j * C, H)]],
                     rows[b].at[pl.ds(0, H)], sems[b])
    pltpu.async_copy(h_hbm.at[srcs_v.at[pl.ds(j * C + H, H)]],
                     rows[b].at[pl.ds(H, H)], gsems[b])

  def gather_wait(j, b):
    pltpu.make_async_copy(h_hbm.at[srcs_v.at[pl.ds(j * C, H)]],
                          rows[b].at[pl.ds(0, H)], sems[b]).wait()
    pltpu.make_async_copy(h_hbm.at[srcs_v.at[pl.ds(j * C + H, H)]],
                          rows[b].at[pl.ds(H, H)], gsems[b]).wait()

  def scatter_start(j, b):
    pltpu.async_copy(rows[b], acc_sh.at[dsts_v.at[j]], ssems[b], add=True)

  def scatter_wait(j, b):
    pltpu.make_async_copy(rows[b], acc_sh.at[dsts_v.at[j]], ssems[b]).wait()

  def counts(j):
    pltpu.sync_copy(ones_v, cnt_sh.at[dsts_v.at[j]], add=True)

  gather_start(0, 0)
  gather_wait(0, 0)
  scatter_start(0, 0)
  gather_start(1, 1)
  counts(0)
  gather_wait(1, 1)
  scatter_start(1, 1)
  scatter_wait(0, 0)
  gather_start(2, 0)
  counts(1)

  @pl.loop(2, K - 1, step=2)
  def _(j0):
    for u in range(2):
      j = j0 + u
      gather_wait(j, u)
      scatter_start(j, u)
      scatter_wait(j - 1, 1 - u)
      gather_start(j + 1, 1 - u)
      counts(j)

  gather_wait(K - 1, 0)
  scatter_start(K - 1, 0)
  scatter_wait(K - 2, 1)
  counts(K - 1)
  scatter_wait(K - 1, 0)

  plsc.subcore_barrier()

  pltpu.sync_copy(acc_sh.at[pl.ds(row0, RPT)], sums_hbm.at[cid, pl.ds(row0, RPT)])
  pltpu.sync_copy(cnt_sh.at[pl.ds(row0, RPT)], cnts_hbm.at[cid, pl.ds(row0, RPT)])


def _sc_aggregate(h, src, dst):
  mesh = plsc.VectorSubcoreMesh(core_axis_name="c", subcore_axis_name="s")
  return pl.kernel(
      _sc_body,
      mesh=mesh,
      out_type=[
          jax.ShapeDtypeStruct((NC, NPAD, D), jnp.float32),
          jax.ShapeDtypeStruct((NC, NPAD), jnp.float32),
      ],
      scratch_types=[
          pltpu.VMEM_SHARED((NPAD, D), jnp.float32),
          pltpu.VMEM_SHARED((NPAD,), jnp.float32),
          pltpu.VMEM((EPW,), jnp.int32),
          pltpu.VMEM((K, C), jnp.int32),
          pltpu.VMEM((C, D), jnp.float32),
          pltpu.VMEM((C, D), jnp.float32),
          pltpu.VMEM((80,), jnp.float32),
          pltpu.VMEM((80,), jnp.float32),
          pltpu.SemaphoreType.DMA,
          pltpu.SemaphoreType.DMA,
          pltpu.SemaphoreType.DMA,
          pltpu.SemaphoreType.DMA,
          pltpu.SemaphoreType.DMA,
          pltpu.SemaphoreType.DMA,
      ],
  )(h, src, dst)


def _tc_body(h_ref, s_ref, c_ref, w_ref, b_ref, o_ref):
  cnt = jnp.maximum(c_ref[0] + c_ref[1], 1.0)
  h_n = (s_ref[0] + s_ref[1]) / cnt
  dn = (((1,), (1,)), ((), ()))
  self_part = lax.dot_general(h_ref[...], w_ref[:, 0:D], dn,
                              preferred_element_type=jnp.float32)
  neigh_part = lax.dot_general(h_n, w_ref[:, D:2 * D], dn,
                               preferred_element_type=jnp.float32)
  o_ref[...] = self_part + neigh_part + b_ref[...]


def _tc_finish(h, sums, cnts, W, b):
  B = 2000
  return pl.pallas_call(
      _tc_body,
      grid=(N // B,),
      in_specs=[
          pl.BlockSpec((B, D), lambda i: (i, 0)),
          pl.BlockSpec((NC, B, D), lambda i: (0, i, 0)),
          pl.BlockSpec((NC, B, 1), lambda i: (0, i, 0)),
          pl.BlockSpec((DOUT, 2 * D), lambda i: (0, 0)),
          pl.BlockSpec((1, DOUT), lambda i: (0, 0)),
      ],
      out_specs=pl.BlockSpec((B, DOUT), lambda i: (i, 0)),
      out_shape=jax.ShapeDtypeStruct((N, DOUT), jnp.float32),
  )(h, sums, cnts, W, b)


@jax.jit
def kernel(h, edge_index, W, b):
  src = edge_index[0].reshape(NW, EPW)
  dst = edge_index[1].reshape(NW, K, C)
  sums, cnts = _sc_aggregate(h, src, dst)
  return _tc_finish(h, sums, cnts.reshape(NC, NPAD, 1), W, b.reshape(1, DOUT))
